# baseline (speedup 1.0000x reference)
import functools

import jax
import jax.numpy as jnp
from jax import lax
from jax.experimental import pallas as pl
from jax.experimental.pallas import tpu as pltpu
from jax.experimental.pallas import tpu_sc as plsc

N = 10000
E = 320000
D = 128

NC = 2
NS = 16
NW = NC * NS
WINE = 96
NWIN = 105
CH = 15
NCH = NWIN // CH
EP = NW * NWIN * WINE
NP = 10112
RPT = NP // NS

_sc_mesh = plsc.VectorSubcoreMesh(core_axis_name="c", subcore_axis_name="s")


@functools.partial(
    pl.kernel,
    out_type=[jax.ShapeDtypeStruct((NP, D), jnp.float32),
              jax.ShapeDtypeStruct((NP, D), jnp.float32)],
    mesh=_sc_mesh,
    scratch_types=[
        pltpu.VMEM((2, CH, WINE), jnp.int32),
        pltpu.VMEM((2, CH, WINE), jnp.int32),
        pltpu.VMEM((3, WINE, D), jnp.float32),
        pltpu.VMEM_SHARED((NP, D), jnp.float32),
        pltpu.SemaphoreType.DMA((3,)),
        pltpu.SemaphoreType.DMA((3,)),
        pltpu.SemaphoreType.DMA((2,)),
    ],
)
def _sc_edge_agg(h_hbm, src_hbm, dst_hbm, zeros_hbm, out0_hbm, out1_hbm,
                 src_v, dst_v, rows_v, agg_sh, gsem, ssem, isem):
    c = lax.axis_index("c")
    s = lax.axis_index("s")
    wid = c * NS + s
    src_w = src_hbm.at[wid]
    dst_w = dst_hbm.at[wid]

    def idx_descs(ch, p):
        return (pltpu.make_async_copy(src_w.at[ch], src_v.at[p], isem.at[p]),
                pltpu.make_async_copy(dst_w.at[ch], dst_v.at[p], isem.at[p]))

    def gd(p, w, b):
        return pltpu.make_async_copy(
            h_hbm.at[src_v.at[p].at[w]], rows_v.at[b], gsem.at[b])

    for d in idx_descs(0, 0):
        d.start()
    for d in idx_descs(1, 1):
        d.start()

    for d in idx_descs(0, 0):
        d.wait()
    gd(0, 0, 0).start()
    gd(0, 1, 1).start()

    pltpu.sync_copy(zeros_hbm.at[pl.ds(s * RPT, RPT)],
                    agg_sh.at[pl.ds(s * RPT, RPT)])
    plsc.subcore_barrier()

    def sd(p, wc, b):
        return pltpu.make_async_copy(
            rows_v.at[b], agg_sh.at[dst_v.at[p].at[wc]], ssem.at[b])

    def body(w, carry):
        b = lax.rem(w, 3)
        ch = lax.div(w, CH)
        p = lax.rem(ch, 2)
        wc = lax.rem(w, CH)
        gd(p, wc, b).wait()
        pltpu.async_copy(rows_v.at[b], agg_sh.at[dst_v.at[p].at[wc]],
                         ssem.at[b], add=True)

        @pl.when(w + 2 < NWIN)
        def _():
            w2 = w + 2
            p2 = lax.rem(lax.div(w2, CH), 2)
            wc2 = lax.rem(w2, CH)

            @pl.when(wc2 == 0)
            def _():
                for d in idx_descs(lax.div(w2, CH), p2):
                    d.wait()

            @pl.when(w >= 1)
            def _():
                wp = w - 1
                sd(lax.rem(lax.div(wp, CH), 2), lax.rem(wp, CH),
                   lax.rem(wp, 3)).wait()

            gd(p2, wc2, lax.rem(w2, 3)).start()

        @pl.when((wc == 0) & (w >= CH) & (ch + 1 < NCH))
        def _():
            for d in idx_descs(ch + 1, lax.rem(ch + 1, 2)):
                d.start()

        return carry

    lax.fori_loop(0, NWIN, body, 0)

    for wl in (NWIN - 3, NWIN - 2, NWIN - 1):
        sd((wl // CH) % 2, wl % CH, wl % 3).wait()

    plsc.subcore_barrier()

    @pl.when(c == 0)
    def _():
        pltpu.sync_copy(agg_sh.at[pl.ds(s * RPT, RPT)],
                        out0_hbm.at[pl.ds(s * RPT, RPT)])

    @pl.when(c == 1)
    def _():
        pltpu.sync_copy(agg_sh.at[pl.ds(s * RPT, RPT)],
                        out1_hbm.at[pl.ds(s * RPT, RPT)])


_ROWS_BLK = 1000
_GRID = N // _ROWS_BLK


def _mlp_body(eps_ref, h_ref, a0_ref, a1_ref, w1_ref, b1_ref, w2_ref, b2_ref,
              o_ref):
    z = h_ref[...] * (1.0 + eps_ref[0]) + a0_ref[...] + a1_ref[...]
    y = jnp.dot(z, w1_ref[...], preferred_element_type=jnp.float32)
    y = jnp.maximum(y + b1_ref[...], 0.0)
    o = jnp.dot(y, w2_ref[...], preferred_element_type=jnp.float32)
    o_ref[...] = jnp.maximum(o + b2_ref[...], 0.0)


_row_spec = pl.BlockSpec((_ROWS_BLK, D), lambda i: (i, 0))
_full_spec = pl.BlockSpec((D, D), lambda i: (0, 0))
_bias_spec = pl.BlockSpec((1, D), lambda i: (0, 0))

_mlp_call = pl.pallas_call(
    _mlp_body,
    grid=(_GRID,),
    in_specs=[
        pl.BlockSpec(memory_space=pltpu.SMEM),
        _row_spec, _row_spec, _row_spec,
        _full_spec, _bias_spec, _full_spec, _bias_spec,
    ],
    out_specs=_row_spec,
    out_shape=jax.ShapeDtypeStruct((N, D), jnp.float32),
)


def _mlp_final_body(eps_ref, h2_ref, a0_ref, a1_ref, w1_ref, b1_ref, w2_ref,
                    b2_ref, x_ref, h1_ref, wf1_ref, bf1_ref, wf2_ref, bf2_ref,
                    o_ref):
    z = h2_ref[...] * (1.0 + eps_ref[0]) + a0_ref[...] + a1_ref[...]
    y = jnp.dot(z, w1_ref[...], preferred_element_type=jnp.float32)
    y = jnp.maximum(y + b1_ref[...], 0.0)
    h3 = jnp.dot(y, w2_ref[...], preferred_element_type=jnp.float32)
    h3 = jnp.maximum(h3 + b2_ref[...], 0.0)
    acc = jnp.dot(x_ref[...], wf1_ref[0:D, :],
                  preferred_element_type=jnp.float32)
    acc += jnp.dot(h1_ref[...], wf1_ref[D:2 * D, :],
                   preferred_element_type=jnp.float32)
    acc += jnp.dot(h2_ref[...], wf1_ref[2 * D:3 * D, :],
                   preferred_element_type=jnp.float32)
    acc += jnp.dot(h3, wf1_ref[3 * D:4 * D, :],
                   preferred_element_type=jnp.float32)
    yf = jnp.maximum(acc + bf1_ref[...], 0.0)
    o_ref[...] = jnp.dot(yf, wf2_ref[...],
                         preferred_element_type=jnp.float32) + bf2_ref[...]


_mlp_final_call = pl.pallas_call(
    _mlp_final_body,
    grid=(_GRID,),
    in_specs=[
        pl.BlockSpec(memory_space=pltpu.SMEM),
        _row_spec, _row_spec, _row_spec,
        _full_spec, _bias_spec, _full_spec, _bias_spec,
        _row_spec, _row_spec,
        pl.BlockSpec((4 * D, D), lambda i: (0, 0)),
        _bias_spec,
        _full_spec,
        _bias_spec,
    ],
    out_specs=_row_spec,
    out_shape=jax.ShapeDtypeStruct((N, D), jnp.float32),
)


def kernel(x, edge_index, eps0, W1_0, b1_0, W2_0, b2_0, eps1, W1_1, b1_1,
           W2_1, b2_1, eps2, W1_2, b1_2, W2_2, b2_2, Wf1, bf1, Wf2, bf2):
    pad = EP - E
    pad_src = (jnp.arange(pad, dtype=jnp.int32) * 13) % N
    pad_dst = N + jnp.arange(pad, dtype=jnp.int32) % (NP - N)
    src = jnp.concatenate([edge_index[0], pad_src]).reshape(NW, NCH, CH, WINE)
    dst = jnp.concatenate([edge_index[1], pad_dst]).reshape(NW, NCH, CH, WINE)
    zeros = jnp.zeros((NP, D), jnp.float32)

    params = [
        (eps0, W1_0, b1_0, W2_0, b2_0),
        (eps1, W1_1, b1_1, W2_1, b2_1),
        (eps2, W1_2, b1_2, W2_2, b2_2),
    ]

    h = x
    feats = [x]
    for eps, W1, b1, W2, b2 in params[:2]:
        p0, p1 = _sc_edge_agg(h, src, dst, zeros)
        h = _mlp_call(eps.reshape(1), h, p0, p1,
                      W1, b1.reshape(1, D), W2, b2.reshape(1, D))
        feats.append(h)

    p0, p1 = _sc_edge_agg(h, src, dst, zeros)
    return _mlp_final_call(eps2.reshape(1), h, p0, p1,
                           W1_2, b1_2.reshape(1, D), W2_2, b2_2.reshape(1, D),
                           feats[0], feats[1],
                           Wf1, bf1.reshape(1, D), Wf2, bf2.reshape(1, D))


# speedup vs baseline: 1.0936x; 1.0936x over previous
import functools

import jax
import jax.numpy as jnp
from jax import lax
from jax.experimental import pallas as pl
from jax.experimental.pallas import tpu as pltpu
from jax.experimental.pallas import tpu_sc as plsc

N = 10000
E = 320000
D = 128

NC = 2
NS = 16
NW = NC * NS
WINE = 128
NWIN = 80
CH = 10
NCH = NWIN // CH
EP = NW * NWIN * WINE
NP = 10112
RPT = NP // NS

_sc_mesh = plsc.VectorSubcoreMesh(core_axis_name="c", subcore_axis_name="s")


@functools.partial(
    pl.kernel,
    out_type=[jax.ShapeDtypeStruct((NP, D), jnp.float32),
              jax.ShapeDtypeStruct((NP, D), jnp.float32)],
    mesh=_sc_mesh,
    scratch_types=[
        pltpu.VMEM((2, CH, WINE), jnp.int32),
        pltpu.VMEM((2, CH, WINE), jnp.int32),
        pltpu.VMEM((2, WINE, D), jnp.float32),
        pltpu.VMEM_SHARED((NP, D), jnp.float32),
        pltpu.SemaphoreType.DMA((2,)),
        pltpu.SemaphoreType.DMA((2,)),
    ],
)
def _sc_edge_agg(h_hbm, src_hbm, dst_hbm, zeros_hbm, out0_hbm, out1_hbm,
                 src_v, dst_v, rows_v, agg_sh, gsem, isem):
    c = lax.axis_index("c")
    s = lax.axis_index("s")
    wid = c * NS + s
    src_w = src_hbm.at[wid]
    dst_w = dst_hbm.at[wid]

    def idx_descs(ch, p):
        return (pltpu.make_async_copy(src_w.at[ch], src_v.at[p], isem.at[p]),
                pltpu.make_async_copy(dst_w.at[ch], dst_v.at[p], isem.at[p]))

    def gd(p, w, b):
        return pltpu.make_async_copy(
            h_hbm.at[src_v.at[p].at[w]], rows_v.at[b], gsem.at[b])

    for d in idx_descs(0, 0):
        d.start()
    for d in idx_descs(1, 1):
        d.start()

    for d in idx_descs(0, 0):
        d.wait()
    gd(0, 0, 0).start()
    gd(0, 1, 1).start()

    pltpu.sync_copy(zeros_hbm.at[pl.ds(s * RPT, RPT)],
                    agg_sh.at[pl.ds(s * RPT, RPT)])
    plsc.subcore_barrier()

    def body(w, carry):
        b = lax.rem(w, 2)
        ch = lax.div(w, CH)
        p = lax.rem(ch, 2)
        wc = lax.rem(w, CH)
        gd(p, wc, b).wait()

        @pl.when(w + 2 < NWIN)
        def _():
            w2 = w + 2
            p2 = lax.rem(lax.div(w2, CH), 2)
            wc2 = lax.rem(w2, CH)

            @pl.when(wc2 == 0)
            def _():
                for d in idx_descs(lax.div(w2, CH), p2):
                    d.wait()

            gd(p2, wc2, lax.rem(w2, 2)).start()

        pltpu.sync_copy(rows_v.at[b], agg_sh.at[dst_v.at[p].at[wc]],
                        add=True)

        @pl.when((wc == 0) & (w >= CH) & (ch + 1 < NCH))
        def _():
            for d in idx_descs(ch + 1, lax.rem(ch + 1, 2)):
                d.start()

        return carry

    lax.fori_loop(0, NWIN, body, 0)

    plsc.subcore_barrier()

    @pl.when(c == 0)
    def _():
        pltpu.sync_copy(agg_sh.at[pl.ds(s * RPT, RPT)],
                        out0_hbm.at[pl.ds(s * RPT, RPT)])

    @pl.when(c == 1)
    def _():
        pltpu.sync_copy(agg_sh.at[pl.ds(s * RPT, RPT)],
                        out1_hbm.at[pl.ds(s * RPT, RPT)])


_ROWS_BLK = 2000
_GRID = N // _ROWS_BLK


def _mlp_body(eps_ref, h_ref, a0_ref, a1_ref, w1_ref, b1_ref, w2_ref, b2_ref,
              o_ref):
    z = h_ref[...] * (1.0 + eps_ref[0]) + a0_ref[...] + a1_ref[...]
    y = jnp.dot(z, w1_ref[...], preferred_element_type=jnp.float32)
    y = jnp.maximum(y + b1_ref[...], 0.0)
    o = jnp.dot(y, w2_ref[...], preferred_element_type=jnp.float32)
    o_ref[...] = jnp.maximum(o + b2_ref[...], 0.0)


_row_spec = pl.BlockSpec((_ROWS_BLK, D), lambda i: (i, 0))
_full_spec = pl.BlockSpec((D, D), lambda i: (0, 0))
_bias_spec = pl.BlockSpec((1, D), lambda i: (0, 0))

_mlp_call = pl.pallas_call(
    _mlp_body,
    grid=(_GRID,),
    in_specs=[
        pl.BlockSpec(memory_space=pltpu.SMEM),
        _row_spec, _row_spec, _row_spec,
        _full_spec, _bias_spec, _full_spec, _bias_spec,
    ],
    out_specs=_row_spec,
    out_shape=jax.ShapeDtypeStruct((N, D), jnp.float32),
)


def _mlp_final_body(eps_ref, h2_ref, a0_ref, a1_ref, w1_ref, b1_ref, w2_ref,
                    b2_ref, x_ref, h1_ref, wf1_ref, bf1_ref, wf2_ref, bf2_ref,
                    o_ref):
    z = h2_ref[...] * (1.0 + eps_ref[0]) + a0_ref[...] + a1_ref[...]
    y = jnp.dot(z, w1_ref[...], preferred_element_type=jnp.float32)
    y = jnp.maximum(y + b1_ref[...], 0.0)
    h3 = jnp.dot(y, w2_ref[...], preferred_element_type=jnp.float32)
    h3 = jnp.maximum(h3 + b2_ref[...], 0.0)
    acc = jnp.dot(x_ref[...], wf1_ref[0:D, :],
                  preferred_element_type=jnp.float32)
    acc += jnp.dot(h1_ref[...], wf1_ref[D:2 * D, :],
                   preferred_element_type=jnp.float32)
    acc += jnp.dot(h2_ref[...], wf1_ref[2 * D:3 * D, :],
                   preferred_element_type=jnp.float32)
    acc += jnp.dot(h3, wf1_ref[3 * D:4 * D, :],
                   preferred_element_type=jnp.float32)
    yf = jnp.maximum(acc + bf1_ref[...], 0.0)
    o_ref[...] = jnp.dot(yf, wf2_ref[...],
                         preferred_element_type=jnp.float32) + bf2_ref[...]


_mlp_final_call = pl.pallas_call(
    _mlp_final_body,
    grid=(_GRID,),
    in_specs=[
        pl.BlockSpec(memory_space=pltpu.SMEM),
        _row_spec, _row_spec, _row_spec,
        _full_spec, _bias_spec, _full_spec, _bias_spec,
        _row_spec, _row_spec,
        pl.BlockSpec((4 * D, D), lambda i: (0, 0)),
        _bias_spec,
        _full_spec,
        _bias_spec,
    ],
    out_specs=_row_spec,
    out_shape=jax.ShapeDtypeStruct((N, D), jnp.float32),
)


def kernel(x, edge_index, eps0, W1_0, b1_0, W2_0, b2_0, eps1, W1_1, b1_1,
           W2_1, b2_1, eps2, W1_2, b1_2, W2_2, b2_2, Wf1, bf1, Wf2, bf2):
    pad = EP - E
    pad_src = (jnp.arange(pad, dtype=jnp.int32) * 13) % N
    pad_dst = N + jnp.arange(pad, dtype=jnp.int32) % (NP - N)
    src = jnp.concatenate([edge_index[0], pad_src]).reshape(NW, NCH, CH, WINE)
    dst = jnp.concatenate([edge_index[1], pad_dst]).reshape(NW, NCH, CH, WINE)
    zeros = jnp.zeros((NP, D), jnp.float32)

    params = [
        (eps0, W1_0, b1_0, W2_0, b2_0),
        (eps1, W1_1, b1_1, W2_1, b2_1),
        (eps2, W1_2, b1_2, W2_2, b2_2),
    ]

    h = x
    feats = [x]
    for eps, W1, b1, W2, b2 in params[:2]:
        p0, p1 = _sc_edge_agg(h, src, dst, zeros)
        h = _mlp_call(eps.reshape(1), h, p0, p1,
                      W1, b1.reshape(1, D), W2, b2.reshape(1, D))
        feats.append(h)

    p0, p1 = _sc_edge_agg(h, src, dst, zeros)
    return _mlp_final_call(eps2.reshape(1), h, p0, p1,
                           W1_2, b1_2.reshape(1, D), W2_2, b2_2.reshape(1, D),
                           feats[0], feats[1],
                           Wf1, bf1.reshape(1, D), Wf2, bf2.reshape(1, D))


# speedup vs baseline: 1.0951x; 1.0014x over previous
import functools

import jax
import jax.numpy as jnp
from jax import lax
from jax.experimental import pallas as pl
from jax.experimental.pallas import tpu as pltpu
from jax.experimental.pallas import tpu_sc as plsc

N = 10000
E = 320000
D = 128

NC = 2
NS = 16
NW = NC * NS
WINE = 128
NWIN = 80
CH = 10
NCH = NWIN // CH
EP = NW * NWIN * WINE
NP = 10112
RPT = NP // NS

_sc_mesh = plsc.VectorSubcoreMesh(core_axis_name="c", subcore_axis_name="s")


@functools.partial(
    pl.kernel,
    out_type=[jax.ShapeDtypeStruct((NP, D), jnp.float32),
              jax.ShapeDtypeStruct((NP, D), jnp.float32)],
    mesh=_sc_mesh,
    scratch_types=[
        pltpu.VMEM((2, CH, WINE), jnp.int32),
        pltpu.VMEM((2, CH, WINE), jnp.int32),
        pltpu.VMEM((2, WINE, D), jnp.float32),
        pltpu.VMEM_SHARED((NP, D), jnp.float32),
        pltpu.SemaphoreType.DMA((2,)),
        pltpu.SemaphoreType.DMA((2,)),
    ],
)
def _sc_edge_agg(h_hbm, src_hbm, dst_hbm, zeros_hbm, out0_hbm, out1_hbm,
                 src_v, dst_v, rows_v, agg_sh, gsem, isem):
    c = lax.axis_index("c")
    s = lax.axis_index("s")
    wid = c * NS + s
    src_w = src_hbm.at[wid]
    dst_w = dst_hbm.at[wid]

    def idx_descs(ch, p):
        return (pltpu.make_async_copy(src_w.at[ch], src_v.at[p], isem.at[p]),
                pltpu.make_async_copy(dst_w.at[ch], dst_v.at[p], isem.at[p]))

    def gd(p, w, b):
        return pltpu.make_async_copy(
            h_hbm.at[src_v.at[p].at[w]], rows_v.at[b], gsem.at[b])

    for d in idx_descs(0, 0):
        d.start()
    for d in idx_descs(1, 1):
        d.start()

    for d in idx_descs(0, 0):
        d.wait()
    gd(0, 0, 0).start()
    gd(0, 1, 1).start()

    pltpu.sync_copy(zeros_hbm.at[pl.ds(s * RPT, RPT)],
                    agg_sh.at[pl.ds(s * RPT, RPT)])
    plsc.subcore_barrier()

    def body(w, carry):
        b = lax.rem(w, 2)
        ch = lax.div(w, CH)
        p = lax.rem(ch, 2)
        wc = lax.rem(w, CH)
        gd(p, wc, b).wait()

        @pl.when(w + 2 < NWIN)
        def _():
            w2 = w + 2
            p2 = lax.rem(lax.div(w2, CH), 2)
            wc2 = lax.rem(w2, CH)

            @pl.when(wc2 == 0)
            def _():
                for d in idx_descs(lax.div(w2, CH), p2):
                    d.wait()

            gd(p2, wc2, lax.rem(w2, 2)).start()

        pltpu.sync_copy(rows_v.at[b], agg_sh.at[dst_v.at[p].at[wc]],
                        add=True)

        @pl.when((wc == 0) & (w >= CH) & (ch + 1 < NCH))
        def _():
            for d in idx_descs(ch + 1, lax.rem(ch + 1, 2)):
                d.start()

        return carry

    lax.fori_loop(0, NWIN, body, 0)

    plsc.subcore_barrier()

    @pl.when(c == 0)
    def _():
        pltpu.sync_copy(agg_sh.at[pl.ds(s * RPT, RPT)],
                        out0_hbm.at[pl.ds(s * RPT, RPT)])

    @pl.when(c == 1)
    def _():
        pltpu.sync_copy(agg_sh.at[pl.ds(s * RPT, RPT)],
                        out1_hbm.at[pl.ds(s * RPT, RPT)])


_ROWS_BLK = 5000
_GRID = N // _ROWS_BLK


def _mlp_body(eps_ref, h_ref, a0_ref, a1_ref, w1_ref, b1_ref, w2_ref, b2_ref,
              o_ref):
    z = h_ref[...] * (1.0 + eps_ref[0]) + a0_ref[...] + a1_ref[...]
    y = jnp.dot(z, w1_ref[...], preferred_element_type=jnp.float32)
    y = jnp.maximum(y + b1_ref[...], 0.0)
    o = jnp.dot(y, w2_ref[...], preferred_element_type=jnp.float32)
    o_ref[...] = jnp.maximum(o + b2_ref[...], 0.0)


_row_spec = pl.BlockSpec((_ROWS_BLK, D), lambda i: (i, 0))
_full_spec = pl.BlockSpec((D, D), lambda i: (0, 0))
_bias_spec = pl.BlockSpec((1, D), lambda i: (0, 0))

_mlp_call = pl.pallas_call(
    _mlp_body,
    grid=(_GRID,),
    in_specs=[
        pl.BlockSpec(memory_space=pltpu.SMEM),
        _row_spec, _row_spec, _row_spec,
        _full_spec, _bias_spec, _full_spec, _bias_spec,
    ],
    out_specs=_row_spec,
    out_shape=jax.ShapeDtypeStruct((N, D), jnp.float32),
)


def _mlp_final_body(eps_ref, h2_ref, a0_ref, a1_ref, w1_ref, b1_ref, w2_ref,
                    b2_ref, x_ref, h1_ref, wf1_ref, bf1_ref, wf2_ref, bf2_ref,
                    o_ref):
    z = h2_ref[...] * (1.0 + eps_ref[0]) + a0_ref[...] + a1_ref[...]
    y = jnp.dot(z, w1_ref[...], preferred_element_type=jnp.float32)
    y = jnp.maximum(y + b1_ref[...], 0.0)
    h3 = jnp.dot(y, w2_ref[...], preferred_element_type=jnp.float32)
    h3 = jnp.maximum(h3 + b2_ref[...], 0.0)
    acc = jnp.dot(x_ref[...], wf1_ref[0:D, :],
                  preferred_element_type=jnp.float32)
    acc += jnp.dot(h1_ref[...], wf1_ref[D:2 * D, :],
                   preferred_element_type=jnp.float32)
    acc += jnp.dot(h2_ref[...], wf1_ref[2 * D:3 * D, :],
                   preferred_element_type=jnp.float32)
    acc += jnp.dot(h3, wf1_ref[3 * D:4 * D, :],
                   preferred_element_type=jnp.float32)
    yf = jnp.maximum(acc + bf1_ref[...], 0.0)
    o_ref[...] = jnp.dot(yf, wf2_ref[...],
                         preferred_element_type=jnp.float32) + bf2_ref[...]


_mlp_final_call = pl.pallas_call(
    _mlp_final_body,
    grid=(_GRID,),
    in_specs=[
        pl.BlockSpec(memory_space=pltpu.SMEM),
        _row_spec, _row_spec, _row_spec,
        _full_spec, _bias_spec, _full_spec, _bias_spec,
        _row_spec, _row_spec,
        pl.BlockSpec((4 * D, D), lambda i: (0, 0)),
        _bias_spec,
        _full_spec,
        _bias_spec,
    ],
    out_specs=_row_spec,
    out_shape=jax.ShapeDtypeStruct((N, D), jnp.float32),
)


def kernel(x, edge_index, eps0, W1_0, b1_0, W2_0, b2_0, eps1, W1_1, b1_1,
           W2_1, b2_1, eps2, W1_2, b1_2, W2_2, b2_2, Wf1, bf1, Wf2, bf2):
    pad = EP - E
    pad_src = (jnp.arange(pad, dtype=jnp.int32) * 13) % N
    pad_dst = N + jnp.arange(pad, dtype=jnp.int32) % (NP - N)
    src = jnp.concatenate([edge_index[0], pad_src]).reshape(NW, NCH, CH, WINE)
    dst = jnp.concatenate([edge_index[1], pad_dst]).reshape(NW, NCH, CH, WINE)
    zeros = jnp.zeros((NP, D), jnp.float32)

    params = [
        (eps0, W1_0, b1_0, W2_0, b2_0),
        (eps1, W1_1, b1_1, W2_1, b2_1),
        (eps2, W1_2, b1_2, W2_2, b2_2),
    ]

    h = x
    feats = [x]
    for eps, W1, b1, W2, b2 in params[:2]:
        p0, p1 = _sc_edge_agg(h, src, dst, zeros)
        h = _mlp_call(eps.reshape(1), h, p0, p1,
                      W1, b1.reshape(1, D), W2, b2.reshape(1, D))
        feats.append(h)

    p0, p1 = _sc_edge_agg(h, src, dst, zeros)
    return _mlp_final_call(eps2.reshape(1), h, p0, p1,
                           W1_2, b1_2.reshape(1, D), W2_2, b2_2.reshape(1, D),
                           feats[0], feats[1],
                           Wf1, bf1.reshape(1, D), Wf2, bf2.reshape(1, D))
